# Initial kernel scaffold; baseline (speedup 1.0000x reference)
import jax
import jax.numpy as jnp
from jax.experimental import pallas as pl


def kernel(S, residue_atom_type, residue_atom_pos, sidechain_chi_angle_atoms, sidechain_chi_mask, sidechain_bonds, sidechain_bonds_mask):
    raise NotImplementedError("write your pallas kernel here")


# speedup vs baseline: 3.5962x; 3.5962x over previous
import functools

import jax
import jax.numpy as jnp
from jax import lax
from jax.experimental import pallas as pl
from jax.experimental.pallas import tpu as pltpu
from jax.experimental.pallas import tpu_sc as plsc

NUM_AA = 26
N_CHANNEL = 14
MAX_CHIS = 4
MAX_BONDS = 11

NC = 2
NS = 16
NW = NC * NS

CHUNK = 512
IDX_W = 128
ROW_W = 96

SECTIONS = (
    (0, N_CHANNEL),
    (16, N_CHANNEL),
    (32, MAX_CHIS * 4),
    (48, MAX_CHIS),
    (56, MAX_BONDS * 2),
    (80, MAX_BONDS),
)


def _sc_gather(n_res):
    assert n_res % (NW * CHUNK) == 0
    per_w = n_res // NW
    n_chunks = per_w // CHUNK

    mesh = plsc.VectorSubcoreMesh(
        core_axis_name="c", subcore_axis_name="s", num_cores=NC, num_subcores=NS
    )

    out_type = tuple(
        jax.ShapeDtypeStruct((n_res * w,), jnp.int32) for _, w in SECTIONS
    )
    scratch = [pltpu.VMEM((CHUNK // IDX_W, IDX_W), jnp.int32),
               pltpu.VMEM((CHUNK, ROW_W), jnp.int32)]
    scratch += [pltpu.VMEM((CHUNK * w + 16,), jnp.int32) for _, w in SECTIONS]
    scratch += [pltpu.SemaphoreType.DMA, pltpu.SemaphoreType.DMA]

    @functools.partial(
        pl.kernel, out_type=out_type, mesh=mesh, scratch_types=scratch,
        compiler_params=pltpu.CompilerParams(use_tc_tiling_on_sc=False),
    )
    def k(s_hbm, tab, o0, o1, o2, o3, o4, o5,
          idx_v, pk, c0, c1, c2, c3, c4, c5, sem_g, sem_w):
        outs = (o0, o1, o2, o3, o4, o5)
        cstg = (c0, c1, c2, c3, c4, c5)
        wid = lax.axis_index("s") * NC + lax.axis_index("c")
        base = wid * per_w

        def chunk_body(ci, carry):
            off = base + ci * CHUNK
            for j in range(CHUNK // IDX_W):
                pltpu.sync_copy(
                    s_hbm.at[pl.ds(off + j * IDX_W, IDX_W)], idx_v.at[j]
                )
            handles = [
                pltpu.async_copy(
                    tab.at[idx_v.at[j]],
                    pk.at[pl.ds(j * IDX_W, IDX_W)],
                    sem_g,
                )
                for j in range(CHUNK // IDX_W)
            ]
            for h in handles:
                h.wait()

            def res_body(r, c):
                for (col, w), stg in zip(SECTIONS, cstg):
                    stg[pl.ds(r * w, 16)] = pk[r, pl.ds(col, 16)]
                    if w > 16:
                        stg[pl.ds(r * w + (w - 16), 16)] = (
                            pk[r, pl.ds(col + (w - 16), 16)]
                        )
                return c

            lax.fori_loop(0, CHUNK, res_body, 0)

            wh = [
                pltpu.async_copy(
                    stg.at[pl.ds(0, CHUNK * w)],
                    out.at[pl.ds(off * w, CHUNK * w)],
                    sem_w,
                )
                for (col, w), stg, out in zip(SECTIONS, cstg, outs)
            ]
            for h in wh:
                h.wait()
            return carry

        lax.fori_loop(0, n_chunks, chunk_body, 0)

    return k


def _pack_tables(parts):
    cols = []
    pos = 0
    for part, (col, w) in zip(parts, SECTIONS):
        if col > pos:
            cols.append(jnp.zeros((NUM_AA, col - pos), jnp.int32))
        cols.append(part)
        pos = col + w
    if pos < ROW_W:
        cols.append(jnp.zeros((NUM_AA, ROW_W - pos), jnp.int32))
    return jnp.concatenate(cols, axis=1)


def kernel(S, residue_atom_type, residue_atom_pos, sidechain_chi_angle_atoms,
           sidechain_chi_mask, sidechain_bonds, sidechain_bonds_mask):
    n_res = S.shape[0]
    tab = _pack_tables((
        residue_atom_type.astype(jnp.int32),
        residue_atom_pos.astype(jnp.int32),
        sidechain_chi_angle_atoms.reshape(NUM_AA, MAX_CHIS * 4).astype(jnp.int32),
        sidechain_chi_mask.astype(jnp.int32),
        sidechain_bonds.reshape(NUM_AA, MAX_BONDS * 2).astype(jnp.int32),
        sidechain_bonds_mask.astype(jnp.int32),
    ))
    o = _sc_gather(n_res)(S, tab)
    atom_type = o[0].reshape(n_res, N_CHANNEL)
    atom_pos = o[1].reshape(n_res, N_CHANNEL)
    chi_angles_atoms = o[2].reshape(n_res, MAX_CHIS, 4)
    chi_mask = o[3].reshape(n_res, MAX_CHIS).astype(jnp.bool_)
    bonds = o[4].reshape(n_res, MAX_BONDS, 2)
    bond_mask = o[5].reshape(n_res, MAX_BONDS).astype(jnp.bool_)
    return (atom_type, atom_pos, chi_angles_atoms, chi_mask, bonds, bond_mask)


# speedup vs baseline: 9.1433x; 2.5425x over previous
import functools

import jax
import jax.numpy as jnp
from jax import lax
from jax.experimental import pallas as pl
from jax.experimental.pallas import tpu as pltpu
from jax.experimental.pallas import tpu_sc as plsc

NUM_AA = 26
N_CHANNEL = 14
MAX_CHIS = 4
MAX_BONDS = 11
N_COLS = 2 * N_CHANNEL + MAX_CHIS * 4 + MAX_CHIS + MAX_BONDS * 2 + MAX_BONDS

NC = 2
NS = 16
NW = NC * NS
L = 16

CHUNK = 1024

PLANE_N = (N_CHANNEL, N_CHANNEL, MAX_CHIS * 4, MAX_CHIS, MAX_BONDS * 2, MAX_BONDS)


def _sc_planar_gather(n_res):
    assert n_res % (NW * CHUNK) == 0
    per_w = n_res // NW
    n_chunks = per_w // CHUNK

    mesh = plsc.VectorSubcoreMesh(
        core_axis_name="c", subcore_axis_name="s", num_cores=NC, num_subcores=NS
    )

    out_type = (
        jax.ShapeDtypeStruct((N_CHANNEL, n_res), jnp.int32),
        jax.ShapeDtypeStruct((N_CHANNEL, n_res), jnp.int32),
        jax.ShapeDtypeStruct((MAX_CHIS, 4, n_res), jnp.int32),
        jax.ShapeDtypeStruct((MAX_CHIS, n_res), jnp.int32),
        jax.ShapeDtypeStruct((MAX_BONDS, 2, n_res), jnp.int32),
        jax.ShapeDtypeStruct((MAX_BONDS, n_res), jnp.int32),
    )
    scratch = [
        pltpu.VMEM((96, NUM_AA), jnp.int32),
        pltpu.VMEM((CHUNK,), jnp.int32),
        pltpu.VMEM((N_COLS, CHUNK), jnp.int32),
        pltpu.SemaphoreType.DMA,
    ]

    @functools.partial(
        pl.kernel, out_type=out_type, mesh=mesh, scratch_types=scratch,
        compiler_params=pltpu.CompilerParams(use_tc_tiling_on_sc=False, needs_layout_passes=False),
    )
    def k(s_hbm, lut_hbm, o0, o1, o2, o3, o4, o5, lut, idx_v, pstg, sem_w):
        outs = (o0, o1, o2, o3, o4, o5)
        wid = lax.axis_index("s") * NC + lax.axis_index("c")
        base = wid * per_w
        pltpu.sync_copy(lut_hbm, lut)

        def chunk_body(ci, carry):
            off = base + ci * CHUNK
            pltpu.sync_copy(s_hbm.at[pl.ds(off, CHUNK)], idx_v)

            def group_body(g, c2):
                s = idx_v[pl.ds(g * L, L)]
                for col in range(N_COLS):
                    val = plsc.load_gather(lut, [jnp.full((L,), col, jnp.int32), s])
                    pstg[col, pl.ds(g * L, L)] = val
                return c2

            lax.fori_loop(0, CHUNK // L, group_body, 0)

            handles = []
            col = 0
            for out, n_planes in zip(outs, PLANE_N):
                for p in range(n_planes):
                    if out.shape == (n_planes, n_res):
                        dst = out.at[p, pl.ds(off, CHUNK)]
                    else:
                        d1 = out.shape[1]
                        dst = out.at[p // d1, p % d1, pl.ds(off, CHUNK)]
                    handles.append(
                        pltpu.async_copy(pstg.at[col], dst, sem_w)
                    )
                    col += 1
            for h in handles:
                h.wait()
            return carry

        lax.fori_loop(0, n_chunks, chunk_body, 0)

    return k


def kernel(S, residue_atom_type, residue_atom_pos, sidechain_chi_angle_atoms,
           sidechain_chi_mask, sidechain_bonds, sidechain_bonds_mask):
    n_res = S.shape[0]
    packed = jnp.concatenate(
        [
            residue_atom_type.astype(jnp.int32),
            residue_atom_pos.astype(jnp.int32),
            sidechain_chi_angle_atoms.reshape(NUM_AA, MAX_CHIS * 4).astype(jnp.int32),
            sidechain_chi_mask.astype(jnp.int32),
            sidechain_bonds.reshape(NUM_AA, MAX_BONDS * 2).astype(jnp.int32),
            sidechain_bonds_mask.astype(jnp.int32),
        ],
        axis=1,
    )
    lut = jnp.zeros((96, NUM_AA), jnp.int32).at[:N_COLS].set(packed.T)

    o = _sc_planar_gather(n_res)(S, lut)
    atom_type = o[0].T
    atom_pos = o[1].T
    chi_angles_atoms = o[2].transpose(2, 0, 1)
    chi_mask = o[3].T.astype(jnp.bool_)
    bonds = o[4].transpose(2, 0, 1)
    bond_mask = o[5].T.astype(jnp.bool_)
    return (atom_type, atom_pos, chi_angles_atoms, chi_mask, bonds, bond_mask)
